# Initial kernel scaffold; baseline (speedup 1.0000x reference)
import jax
import jax.numpy as jnp
from jax.experimental import pallas as pl


def kernel(h, rel_x, r_feat, edge_feat, edge_index, params):
    raise NotImplementedError("write your pallas kernel here")


# speedup vs baseline: 21.3007x; 21.3007x over previous
import functools

import jax
import jax.numpy as jnp
import numpy as np
from jax import lax
from jax.experimental import pallas as pl
from jax.experimental.pallas import tpu as pltpu
from jax.experimental.pallas import tpu_sc as plsc

N_NODES = 10000
N_EDGES = 320000
D = 128
HID = 128
OUT = 128
H = 16
HEAD_DIM = 8
INV_SQRT_HD = 1.0 / np.sqrt(HEAD_DIM)

EW = 128

NC = 2
NS = 16
NW = NC * NS
EPW = N_EDGES // NW
BLK = 80
N_PAD = 10240
NPT = N_PAD // NS


def _ln_relu(y, g, be):
    mu = jnp.mean(y, axis=-1, keepdims=True)
    var = jnp.mean((y - mu) * (y - mu), axis=-1, keepdims=True)
    y = (y - mu) * lax.rsqrt(var + 1e-5) * g + be
    return jnp.maximum(y, 0.0)


def _phase_a_body(h_ref, w1_ref, b1_ref, g_ref, be_ref, w2_ref, b2_ref, out_ref):
    x = h_ref[...]
    y = jnp.dot(x, w1_ref[...], preferred_element_type=jnp.float32) + b1_ref[...]
    y = _ln_relu(y, g_ref[...], be_ref[...])
    q = jnp.dot(y, w2_ref[...], preferred_element_type=jnp.float32) + b2_ref[...]
    out_ref[:, :D] = x
    out_ref[:, D:] = q


def _phase_a(h, pq):
    bn = 2000
    grid = (N_NODES // bn,)
    full = lambda shape: pl.BlockSpec(shape, lambda i: (0, 0))
    return pl.pallas_call(
        _phase_a_body,
        grid=grid,
        in_specs=[
            pl.BlockSpec((bn, D), lambda i: (i, 0)),
            full((D, HID)), full((1, HID)), full((1, HID)), full((1, HID)),
            full((HID, OUT)), full((1, OUT)),
        ],
        out_specs=pl.BlockSpec((bn, 2 * D), lambda i: (i, 0)),
        out_shape=jax.ShapeDtypeStruct((N_NODES, 2 * D), jnp.float32),
    )(h, pq['W1'], pq['b1'].reshape(1, -1), pq['g'].reshape(1, -1),
      pq['be'].reshape(1, -1), pq['W2'], pq['b2'].reshape(1, -1))


def _phase_b_body(table_hbm, h_hbm, dst_hbm, src_hbm, g1_hbm, g2_hbm,
                  idxd, idxs, buf1, buf2, sem1, sem2):
    wid = lax.axis_index("s") * NC + lax.axis_index("c")
    base0 = wid * EPW

    def step(i, _):
        base = base0 + i * BLK
        pltpu.sync_copy(dst_hbm.at[pl.ds(base, BLK)], idxd)
        pltpu.sync_copy(src_hbm.at[pl.ds(base, BLK)], idxs)
        c1 = pltpu.async_copy(table_hbm.at[idxd], buf1, sem1)
        c2 = pltpu.async_copy(h_hbm.at[idxs], buf2, sem2)
        c1.wait()
        c2.wait()
        pltpu.sync_copy(buf1, g1_hbm.at[pl.ds(base, BLK)])
        pltpu.sync_copy(buf2, g2_hbm.at[pl.ds(base, BLK)])
        return 0

    lax.fori_loop(0, EPW // BLK, step, 0)


def _phase_b(table, h, dst, src):
    mesh = plsc.VectorSubcoreMesh(core_axis_name="c", subcore_axis_name="s")
    k = pl.kernel(
        _phase_b_body,
        out_type=[jax.ShapeDtypeStruct((N_EDGES, 2 * D), jnp.float32),
                  jax.ShapeDtypeStruct((N_EDGES, D), jnp.float32)],
        mesh=mesh,
        scratch_types=[
            pltpu.VMEM((BLK,), jnp.int32),
            pltpu.VMEM((BLK,), jnp.int32),
            pltpu.VMEM((BLK, 2 * D), jnp.float32),
            pltpu.VMEM((BLK, D), jnp.float32),
            pltpu.SemaphoreType.DMA,
            pltpu.SemaphoreType.DMA,
        ],
    )
    return k(table, h, dst, src)


def _phase_c_body(g1_ref, g2_ref, ef_ref, rf_ref, rx_ref,
                  w1k_ref, gk_ref, bek_ref, w2k_ref,
                  w1v_ref, gv_ref, bev_ref, w2v_ref,
                  b1k_ref, b2k_ref, b1v_ref, b2v_ref,
                  eww_ref, ewb_ref, hsel_ref, rsel_ref, out_ref):
    hi = g1_ref[:, :D]
    qd = g1_ref[:, D:]
    hj = g2_ref[...]
    ef = ef_ref[...]
    rf = rf_ref[...]

    def mlp_pre(w1_ref, b1, g, be):
        y = (jnp.dot(ef, w1_ref[0:16, :], preferred_element_type=jnp.float32)
             + jnp.dot(rf, w1_ref[16:32, :], preferred_element_type=jnp.float32)
             + jnp.dot(hi, w1_ref[32:160, :], preferred_element_type=jnp.float32)
             + jnp.dot(hj, w1_ref[160:288, :], preferred_element_type=jnp.float32)
             + b1)
        return _ln_relu(y, g, be)

    yk = mlp_pre(w1k_ref, b1k_ref[...], gk_ref[...], bek_ref[...])
    k = jnp.dot(yk, w2k_ref[...], preferred_element_type=jnp.float32) + b2k_ref[...]
    logits = jnp.dot(qd * k, hsel_ref[...], preferred_element_type=jnp.float32) * INV_SQRT_HD
    ex = jnp.exp(logits)

    yv = mlp_pre(w1v_ref, b1v_ref[...], gv_ref[...], bev_ref[...])
    vs = jnp.dot(yv, w2v_ref[...], preferred_element_type=jnp.float32) + b2v_ref[...]
    ew = jax.nn.sigmoid(jnp.sum(rf * eww_ref[...], axis=-1, keepdims=True)
                        + ewb_ref[:, 0:1])
    prod = ex * (vs * ew)

    rxb = jnp.dot(rx_ref[...], rsel_ref[...], preferred_element_type=jnp.float32)
    prod3 = jnp.concatenate([prod, prod, prod], axis=-1) * rxb
    out_ref[...] = jnp.concatenate(
        [ex, prod3, jnp.zeros_like(ex), prod3 * 0.0], axis=-1)


def _phase_c(g1, g2, edge_feat, r_feat, rx8, pk, pv, pew):
    be_ = 512
    grid = (N_EDGES // be_,)
    data = lambda w: pl.BlockSpec((be_, w), lambda i: (i, 0))
    full = lambda shape: pl.BlockSpec(shape, lambda i: (0, 0))

    hsel = np.repeat(np.eye(H, dtype=np.float32), HEAD_DIM, axis=0)
    rsel = np.zeros((8, 3 * H), np.float32)
    for d_ in range(3):
        rsel[d_, d_ * H:(d_ + 1) * H] = 1.0

    kin = 2 * D + 32
    return pl.pallas_call(
        _phase_c_body,
        grid=grid,
        in_specs=[
            data(2 * D), data(D), data(16), data(16), data(8),
            full((kin, HID)), full((1, HID)), full((1, HID)), full((HID, OUT)),
            full((kin, HID)), full((1, HID)), full((1, HID)), full((HID, H)),
            full((1, HID)), full((1, OUT)), full((1, HID)), full((1, H)),
            full((1, 16)), full((1, 16)), full((D, H)), full((8, 3 * H)),
        ],
        out_specs=pl.BlockSpec((be_, EW), lambda i: (i, 0)),
        out_shape=jax.ShapeDtypeStruct((N_EDGES, EW), jnp.float32),
    )(g1, g2, edge_feat, r_feat, rx8,
      pk['W1'], pk['g'].reshape(1, -1), pk['be'].reshape(1, -1), pk['W2'],
      pv['W1'], pv['g'].reshape(1, -1), pv['be'].reshape(1, -1), pv['W2'],
      pk['b1'].reshape(1, -1), pk['b2'].reshape(1, -1),
      pv['b1'].reshape(1, -1), pv['b2'].reshape(1, -1),
      pew['W'].reshape(1, -1), jnp.broadcast_to(pew['b'].reshape(1, 1), (1, 16)),
      jnp.asarray(hsel), jnp.asarray(rsel))


def _phase_d_body(vals_hbm, dst_hbm, zeros_hbm, out_hbm, idx, buf, acc_sh, sem):
    c = lax.axis_index("c")
    s = lax.axis_index("s")
    wid = s * NC + c
    base0 = wid * EPW

    pltpu.sync_copy(zeros_hbm.at[pl.ds(0, NPT)], acc_sh.at[pl.ds(s * NPT, NPT)])
    plsc.subcore_barrier()

    def step(i, _):
        base = base0 + i * BLK
        pltpu.sync_copy(dst_hbm.at[pl.ds(base, BLK)], idx.at[0])
        c1 = pltpu.async_copy(vals_hbm.at[pl.ds(base, BLK)], buf, sem)
        c1.wait()
        pltpu.sync_copy(buf, acc_sh.at[idx.at[0]], add=True)
        return 0

    lax.fori_loop(0, EPW // BLK, step, 0)

    plsc.subcore_barrier()
    pltpu.sync_copy(acc_sh.at[pl.ds(s * NPT, NPT)],
                    out_hbm.at[c, pl.ds(s * NPT, NPT)])


def _phase_d(vals, dst, zeros):
    mesh = plsc.VectorSubcoreMesh(core_axis_name="c", subcore_axis_name="s")
    k = pl.kernel(
        _phase_d_body,
        out_type=jax.ShapeDtypeStruct((NC, N_PAD, EW), jnp.float32),
        mesh=mesh,
        scratch_types=[
            pltpu.VMEM((1, BLK), jnp.int32),
            pltpu.VMEM((BLK, EW), jnp.float32),
            pltpu.MemorySpace.VMEM_SHARED((N_PAD, EW), jnp.float32),
            pltpu.SemaphoreType.DMA,
        ],
    )
    return k(vals, dst, zeros)


def _phase_e_body(acc_ref, out_ref):
    a = acc_ref[0] + acc_ref[1]
    inv = 1.0 / ((a[:, :H] + 1e-16) * float(H))
    o0 = jnp.sum(a[:, H:2 * H] * inv, axis=-1, keepdims=True)
    o1 = jnp.sum(a[:, 2 * H:3 * H] * inv, axis=-1, keepdims=True)
    o2 = jnp.sum(a[:, 3 * H:4 * H] * inv, axis=-1, keepdims=True)
    out_ref[...] = jnp.concatenate([o0, o1, o2, o0 * 0.0], axis=-1)


def _phase_e(acc):
    bn = 2000
    grid = (N_NODES // bn,)
    return pl.pallas_call(
        _phase_e_body,
        grid=grid,
        in_specs=[pl.BlockSpec((NC, bn, EW), lambda i: (0, i, 0))],
        out_specs=pl.BlockSpec((bn, 4), lambda i: (i, 0)),
        out_shape=jax.ShapeDtypeStruct((N_NODES, 4), jnp.float32),
    )(acc)


def kernel(h, rel_x, r_feat, edge_feat, edge_index, params):
    src = edge_index[0]
    dst = edge_index[1]
    table = _phase_a(h, params['xq'])
    g1, g2 = _phase_b(table, h, dst, src)
    rx8 = jnp.concatenate(
        [rel_x, jnp.zeros((N_EDGES, 5), jnp.float32)], axis=1)
    ev = _phase_c(g1, g2, edge_feat, r_feat, rx8,
                  params['xk'], params['xv'], params['ew'])
    zeros = jnp.zeros((NPT, EW), jnp.float32)
    acc = _phase_d(ev, dst, zeros)
    out4 = _phase_e(acc)
    return out4[:, :3]


# speedup vs baseline: 21.6035x; 1.0142x over previous
import functools

import jax
import jax.numpy as jnp
import numpy as np
from jax import lax
from jax.experimental import pallas as pl
from jax.experimental.pallas import tpu as pltpu
from jax.experimental.pallas import tpu_sc as plsc

N_NODES = 10000
N_EDGES = 320000
D = 128
HID = 128
OUT = 128
H = 16
HEAD_DIM = 8
INV_SQRT_HD = 1.0 / np.sqrt(HEAD_DIM)

EW = 128

NC = 2
NS = 16
NW = NC * NS
EPW = N_EDGES // NW
BLK = 80
N_PAD = 10240
NPT = N_PAD // NS


def _ln_relu(y, g, be):
    mu = jnp.mean(y, axis=-1, keepdims=True)
    var = jnp.mean((y - mu) * (y - mu), axis=-1, keepdims=True)
    y = (y - mu) * lax.rsqrt(var + 1e-5) * g + be
    return jnp.maximum(y, 0.0)


def _phase_a_body(h_ref, w1_ref, b1_ref, g_ref, be_ref, w2_ref, b2_ref, out_ref):
    x = h_ref[...]
    y = jnp.dot(x, w1_ref[...], preferred_element_type=jnp.float32) + b1_ref[...]
    y = _ln_relu(y, g_ref[...], be_ref[...])
    q = jnp.dot(y, w2_ref[...], preferred_element_type=jnp.float32) + b2_ref[...]
    out_ref[:, :D] = x
    out_ref[:, D:] = q


def _phase_a(h, pq):
    bn = 2000
    grid = (N_NODES // bn,)
    full = lambda shape: pl.BlockSpec(shape, lambda i: (0, 0))
    return pl.pallas_call(
        _phase_a_body,
        grid=grid,
        in_specs=[
            pl.BlockSpec((bn, D), lambda i: (i, 0)),
            full((D, HID)), full((1, HID)), full((1, HID)), full((1, HID)),
            full((HID, OUT)), full((1, OUT)),
        ],
        out_specs=pl.BlockSpec((bn, 2 * D), lambda i: (i, 0)),
        out_shape=jax.ShapeDtypeStruct((N_NODES, 2 * D), jnp.float32),
    )(h, pq['W1'], pq['b1'].reshape(1, -1), pq['g'].reshape(1, -1),
      pq['be'].reshape(1, -1), pq['W2'], pq['b2'].reshape(1, -1))


def _phase_b_body(table_hbm, h_hbm, dst_hbm, src_hbm, g1_hbm, g2_hbm,
                  idxd, idxs, buf1, buf2, sem1, sem2):
    wid = lax.axis_index("s") * NC + lax.axis_index("c")
    base0 = wid * EPW
    nblk = EPW // BLK

    def fetch(i, sl):
        base = base0 + i * BLK
        pltpu.sync_copy(dst_hbm.at[pl.ds(base, BLK)], idxd.at[sl])
        pltpu.sync_copy(src_hbm.at[pl.ds(base, BLK)], idxs.at[sl])
        pltpu.async_copy(table_hbm.at[idxd.at[sl]], buf1.at[sl], sem1.at[sl])
        pltpu.async_copy(h_hbm.at[idxs.at[sl]], buf2.at[sl], sem2.at[sl])

    def wait_slot(sl):
        pltpu.make_async_copy(
            table_hbm.at[idxd.at[sl]], buf1.at[sl], sem1.at[sl]).wait()
        pltpu.make_async_copy(
            h_hbm.at[idxs.at[sl]], buf2.at[sl], sem2.at[sl]).wait()

    fetch(0, 0)

    def step(i, _):
        sl = lax.rem(i, 2)
        lax.cond(i + 1 < nblk,
                 lambda: fetch(i + 1, 1 - sl),
                 lambda: None)
        wait_slot(sl)
        base = base0 + i * BLK
        pltpu.sync_copy(buf1.at[sl], g1_hbm.at[pl.ds(base, BLK)])
        pltpu.sync_copy(buf2.at[sl], g2_hbm.at[pl.ds(base, BLK)])
        return 0

    lax.fori_loop(0, nblk, step, 0)


def _phase_b(table, h, dst, src):
    mesh = plsc.VectorSubcoreMesh(core_axis_name="c", subcore_axis_name="s")
    k = pl.kernel(
        _phase_b_body,
        out_type=[jax.ShapeDtypeStruct((N_EDGES, 2 * D), jnp.float32),
                  jax.ShapeDtypeStruct((N_EDGES, D), jnp.float32)],
        mesh=mesh,
        scratch_types=[
            pltpu.VMEM((2, BLK), jnp.int32),
            pltpu.VMEM((2, BLK), jnp.int32),
            pltpu.VMEM((2, BLK, 2 * D), jnp.float32),
            pltpu.VMEM((2, BLK, D), jnp.float32),
            pltpu.SemaphoreType.DMA((2,)),
            pltpu.SemaphoreType.DMA((2,)),
        ],
    )
    return k(table, h, dst, src)


def _phase_c_body(g1_ref, g2_ref, ef_ref, rf_ref, rx_ref,
                  w1k_ref, gk_ref, bek_ref, w2k_ref,
                  w1v_ref, gv_ref, bev_ref, w2v_ref,
                  b1k_ref, b2k_ref, b1v_ref, b2v_ref,
                  eww_ref, ewb_ref, hsel_ref, rsel_ref, out_ref):
    hi = g1_ref[:, :D]
    qd = g1_ref[:, D:]
    hj = g2_ref[...]
    ef = ef_ref[...]
    rf = rf_ref[...]

    def mlp_pre(w1_ref, b1, g, be):
        y = (jnp.dot(ef, w1_ref[0:16, :], preferred_element_type=jnp.float32)
             + jnp.dot(rf, w1_ref[16:32, :], preferred_element_type=jnp.float32)
             + jnp.dot(hi, w1_ref[32:160, :], preferred_element_type=jnp.float32)
             + jnp.dot(hj, w1_ref[160:288, :], preferred_element_type=jnp.float32)
             + b1)
        return _ln_relu(y, g, be)

    yk = mlp_pre(w1k_ref, b1k_ref[...], gk_ref[...], bek_ref[...])
    k = jnp.dot(yk, w2k_ref[...], preferred_element_type=jnp.float32) + b2k_ref[...]
    logits = jnp.dot(qd * k, hsel_ref[...], preferred_element_type=jnp.float32) * INV_SQRT_HD
    ex = jnp.exp(logits)

    yv = mlp_pre(w1v_ref, b1v_ref[...], gv_ref[...], bev_ref[...])
    vs = jnp.dot(yv, w2v_ref[...], preferred_element_type=jnp.float32) + b2v_ref[...]
    ew = jax.nn.sigmoid(jnp.sum(rf * eww_ref[...], axis=-1, keepdims=True)
                        + ewb_ref[:, 0:1])
    prod = ex * (vs * ew)

    rxb = jnp.dot(rx_ref[...], rsel_ref[...], preferred_element_type=jnp.float32)
    prod3 = jnp.concatenate([prod, prod, prod], axis=-1) * rxb
    out_ref[...] = jnp.concatenate(
        [ex, prod3, jnp.zeros_like(ex), prod3 * 0.0], axis=-1)


def _phase_c(g1, g2, edge_feat, r_feat, rx8, pk, pv, pew):
    be_ = 512
    grid = (N_EDGES // be_,)
    data = lambda w: pl.BlockSpec((be_, w), lambda i: (i, 0))
    full = lambda shape: pl.BlockSpec(shape, lambda i: (0, 0))

    hsel = np.repeat(np.eye(H, dtype=np.float32), HEAD_DIM, axis=0)
    rsel = np.zeros((8, 3 * H), np.float32)
    for d_ in range(3):
        rsel[d_, d_ * H:(d_ + 1) * H] = 1.0

    kin = 2 * D + 32
    return pl.pallas_call(
        _phase_c_body,
        grid=grid,
        in_specs=[
            data(2 * D), data(D), data(16), data(16), data(8),
            full((kin, HID)), full((1, HID)), full((1, HID)), full((HID, OUT)),
            full((kin, HID)), full((1, HID)), full((1, HID)), full((HID, H)),
            full((1, HID)), full((1, OUT)), full((1, HID)), full((1, H)),
            full((1, 16)), full((1, 16)), full((D, H)), full((8, 3 * H)),
        ],
        out_specs=pl.BlockSpec((be_, EW), lambda i: (i, 0)),
        out_shape=jax.ShapeDtypeStruct((N_EDGES, EW), jnp.float32),
    )(g1, g2, edge_feat, r_feat, rx8,
      pk['W1'], pk['g'].reshape(1, -1), pk['be'].reshape(1, -1), pk['W2'],
      pv['W1'], pv['g'].reshape(1, -1), pv['be'].reshape(1, -1), pv['W2'],
      pk['b1'].reshape(1, -1), pk['b2'].reshape(1, -1),
      pv['b1'].reshape(1, -1), pv['b2'].reshape(1, -1),
      pew['W'].reshape(1, -1), jnp.broadcast_to(pew['b'].reshape(1, 1), (1, 16)),
      jnp.asarray(hsel), jnp.asarray(rsel))


def _phase_d_body(vals_hbm, dst_hbm, zeros_hbm, out_hbm, idx, buf, acc_sh, sem):
    c = lax.axis_index("c")
    s = lax.axis_index("s")
    wid = s * NC + c
    base0 = wid * EPW

    pltpu.sync_copy(zeros_hbm.at[pl.ds(0, NPT)], acc_sh.at[pl.ds(s * NPT, NPT)])
    plsc.subcore_barrier()

    def step(i, _):
        base = base0 + i * BLK
        pltpu.sync_copy(dst_hbm.at[pl.ds(base, BLK)], idx.at[0])
        c1 = pltpu.async_copy(vals_hbm.at[pl.ds(base, BLK)], buf, sem)
        c1.wait()
        pltpu.sync_copy(buf, acc_sh.at[idx.at[0]], add=True)
        return 0

    lax.fori_loop(0, EPW // BLK, step, 0)

    plsc.subcore_barrier()
    pltpu.sync_copy(acc_sh.at[pl.ds(s * NPT, NPT)],
                    out_hbm.at[c, pl.ds(s * NPT, NPT)])


def _phase_d(vals, dst, zeros):
    mesh = plsc.VectorSubcoreMesh(core_axis_name="c", subcore_axis_name="s")
    k = pl.kernel(
        _phase_d_body,
        out_type=jax.ShapeDtypeStruct((NC, N_PAD, EW), jnp.float32),
        mesh=mesh,
        scratch_types=[
            pltpu.VMEM((1, BLK), jnp.int32),
            pltpu.VMEM((BLK, EW), jnp.float32),
            pltpu.MemorySpace.VMEM_SHARED((N_PAD, EW), jnp.float32),
            pltpu.SemaphoreType.DMA,
        ],
    )
    return k(vals, dst, zeros)


def _phase_e_body(acc_ref, out_ref):
    a = acc_ref[0] + acc_ref[1]
    inv = 1.0 / ((a[:, :H] + 1e-16) * float(H))
    o0 = jnp.sum(a[:, H:2 * H] * inv, axis=-1, keepdims=True)
    o1 = jnp.sum(a[:, 2 * H:3 * H] * inv, axis=-1, keepdims=True)
    o2 = jnp.sum(a[:, 3 * H:4 * H] * inv, axis=-1, keepdims=True)
    out_ref[...] = jnp.concatenate([o0, o1, o2, o0 * 0.0], axis=-1)


def _phase_e(acc):
    bn = 2000
    grid = (N_NODES // bn,)
    return pl.pallas_call(
        _phase_e_body,
        grid=grid,
        in_specs=[pl.BlockSpec((NC, bn, EW), lambda i: (0, i, 0))],
        out_specs=pl.BlockSpec((bn, 4), lambda i: (i, 0)),
        out_shape=jax.ShapeDtypeStruct((N_NODES, 4), jnp.float32),
    )(acc)


def kernel(h, rel_x, r_feat, edge_feat, edge_index, params):
    src = edge_index[0]
    dst = edge_index[1]
    table = _phase_a(h, params['xq'])
    g1, g2 = _phase_b(table, h, dst, src)
    rx8 = jnp.concatenate(
        [rel_x, jnp.zeros((N_EDGES, 5), jnp.float32)], axis=1)
    ev = _phase_c(g1, g2, edge_feat, r_feat, rx8,
                  params['xk'], params['xv'], params['ew'])
    zeros = jnp.zeros((NPT, EW), jnp.float32)
    acc = _phase_d(ev, dst, zeros)
    out4 = _phase_e(acc)
    return out4[:, :3]


# speedup vs baseline: 22.8829x; 1.0592x over previous
import functools

import jax
import jax.numpy as jnp
import numpy as np
from jax import lax
from jax.experimental import pallas as pl
from jax.experimental.pallas import tpu as pltpu
from jax.experimental.pallas import tpu_sc as plsc

N_NODES = 10000
N_EDGES = 320000
D = 128
HID = 128
OUT = 128
H = 16
HEAD_DIM = 8
INV_SQRT_HD = 1.0 / np.sqrt(HEAD_DIM)

EW = 128

NC = 2
NS = 16
NW = NC * NS
EPW = N_EDGES // NW
BLK = 80
N_PAD = 10240
NPT = N_PAD // NS


def _ln_relu(y, g, be):
    mu = jnp.mean(y, axis=-1, keepdims=True)
    var = jnp.mean((y - mu) * (y - mu), axis=-1, keepdims=True)
    y = (y - mu) * lax.rsqrt(var + 1e-5) * g + be
    return jnp.maximum(y, 0.0)


def _phase_a_body(h_ref, w1_ref, b1_ref, g_ref, be_ref, w2_ref, b2_ref, out_ref):
    x = h_ref[...]
    y = jnp.dot(x, w1_ref[...], preferred_element_type=jnp.float32) + b1_ref[...]
    y = _ln_relu(y, g_ref[...], be_ref[...])
    q = jnp.dot(y, w2_ref[...], preferred_element_type=jnp.float32) + b2_ref[...]
    out_ref[:, :D] = x
    out_ref[:, D:] = q


def _phase_a(h, pq):
    bn = 2000
    grid = (N_NODES // bn,)
    full = lambda shape: pl.BlockSpec(shape, lambda i: (0, 0))
    return pl.pallas_call(
        _phase_a_body,
        grid=grid,
        in_specs=[
            pl.BlockSpec((bn, D), lambda i: (i, 0)),
            full((D, HID)), full((1, HID)), full((1, HID)), full((1, HID)),
            full((HID, OUT)), full((1, OUT)),
        ],
        out_specs=pl.BlockSpec((bn, 2 * D), lambda i: (i, 0)),
        out_shape=jax.ShapeDtypeStruct((N_NODES, 2 * D), jnp.float32),
    )(h, pq['W1'], pq['b1'].reshape(1, -1), pq['g'].reshape(1, -1),
      pq['be'].reshape(1, -1), pq['W2'], pq['b2'].reshape(1, -1))


def _phase_b_body(table_hbm, h_hbm, dst_hbm, src_hbm, g1_hbm, g2_hbm,
                  idxd, idxs, buf1, buf2, sem1, sem2):
    wid = lax.axis_index("s") * NC + lax.axis_index("c")
    base0 = wid * EPW
    nblk = EPW // BLK

    def fetch(i, sl):
        base = base0 + i * BLK
        pltpu.sync_copy(dst_hbm.at[pl.ds(base, BLK)], idxd.at[sl])
        pltpu.sync_copy(src_hbm.at[pl.ds(base, BLK)], idxs.at[sl])
        pltpu.async_copy(table_hbm.at[idxd.at[sl]], buf1.at[sl], sem1.at[sl])
        pltpu.async_copy(h_hbm.at[idxs.at[sl]], buf2.at[sl], sem2.at[sl])

    def wait_slot(sl):
        pltpu.make_async_copy(
            table_hbm.at[idxd.at[sl]], buf1.at[sl], sem1.at[sl]).wait()
        pltpu.make_async_copy(
            h_hbm.at[idxs.at[sl]], buf2.at[sl], sem2.at[sl]).wait()

    fetch(0, 0)

    def step(i, _):
        sl = lax.rem(i, 2)
        lax.cond(i + 1 < nblk,
                 lambda: fetch(i + 1, 1 - sl),
                 lambda: None)
        wait_slot(sl)
        base = base0 + i * BLK
        pltpu.sync_copy(buf1.at[sl], g1_hbm.at[pl.ds(base, BLK)])
        pltpu.sync_copy(buf2.at[sl], g2_hbm.at[pl.ds(base, BLK)])
        return 0

    lax.fori_loop(0, nblk, step, 0)


def _phase_b(table, h, dst, src):
    mesh = plsc.VectorSubcoreMesh(core_axis_name="c", subcore_axis_name="s")
    k = pl.kernel(
        _phase_b_body,
        out_type=[jax.ShapeDtypeStruct((N_EDGES, 2 * D), jnp.float32),
                  jax.ShapeDtypeStruct((N_EDGES, D), jnp.float32)],
        mesh=mesh,
        scratch_types=[
            pltpu.VMEM((2, BLK), jnp.int32),
            pltpu.VMEM((2, BLK), jnp.int32),
            pltpu.VMEM((2, BLK, 2 * D), jnp.float32),
            pltpu.VMEM((2, BLK, D), jnp.float32),
            pltpu.SemaphoreType.DMA((2,)),
            pltpu.SemaphoreType.DMA((2,)),
        ],
    )
    return k(table, h, dst, src)


def _phase_c_body(g1_ref, g2_ref, ef_ref, rf_ref, rx_ref,
                  w1k_ref, gk_ref, bek_ref, w2k_ref,
                  w1v_ref, gv_ref, bev_ref, w2v_ref,
                  b1k_ref, b2k_ref, b1v_ref, b2v_ref,
                  eww_ref, ewb_ref, hsel_ref, rsel_ref, out_ref):
    hi = g1_ref[:, :D]
    qd = g1_ref[:, D:]
    hj = g2_ref[...]
    ef = ef_ref[...]
    rf = rf_ref[...]

    def mlp_pre(w1_ref, b1, g, be):
        y = (jnp.dot(ef, w1_ref[0:16, :], preferred_element_type=jnp.float32)
             + jnp.dot(rf, w1_ref[16:32, :], preferred_element_type=jnp.float32)
             + jnp.dot(hi, w1_ref[32:160, :], preferred_element_type=jnp.float32)
             + jnp.dot(hj, w1_ref[160:288, :], preferred_element_type=jnp.float32)
             + b1)
        return _ln_relu(y, g, be)

    yk = mlp_pre(w1k_ref, b1k_ref[...], gk_ref[...], bek_ref[...])
    k = jnp.dot(yk, w2k_ref[...], preferred_element_type=jnp.float32) + b2k_ref[...]
    logits = jnp.dot(qd * k, hsel_ref[...], preferred_element_type=jnp.float32) * INV_SQRT_HD
    ex = jnp.exp(logits)

    yv = mlp_pre(w1v_ref, b1v_ref[...], gv_ref[...], bev_ref[...])
    vs = jnp.dot(yv, w2v_ref[...], preferred_element_type=jnp.float32) + b2v_ref[...]
    ew = jax.nn.sigmoid(jnp.sum(rf * eww_ref[...], axis=-1, keepdims=True)
                        + ewb_ref[:, 0:1])
    prod = ex * (vs * ew)

    rxb = jnp.dot(rx_ref[...], rsel_ref[...], preferred_element_type=jnp.float32)
    prod3 = jnp.concatenate([prod, prod, prod], axis=-1) * rxb
    out_ref[...] = jnp.concatenate(
        [ex, prod3, jnp.zeros_like(ex), prod3 * 0.0], axis=-1)


def _phase_c(g1, g2, edge_feat, r_feat, rx8, pk, pv, pew):
    be_ = 512
    grid = (N_EDGES // be_,)
    data = lambda w: pl.BlockSpec((be_, w), lambda i: (i, 0))
    full = lambda shape: pl.BlockSpec(shape, lambda i: (0, 0))

    hsel = np.repeat(np.eye(H, dtype=np.float32), HEAD_DIM, axis=0)
    rsel = np.zeros((8, 3 * H), np.float32)
    for d_ in range(3):
        rsel[d_, d_ * H:(d_ + 1) * H] = 1.0

    kin = 2 * D + 32
    return pl.pallas_call(
        _phase_c_body,
        grid=grid,
        in_specs=[
            data(2 * D), data(D), data(16), data(16), data(8),
            full((kin, HID)), full((1, HID)), full((1, HID)), full((HID, OUT)),
            full((kin, HID)), full((1, HID)), full((1, HID)), full((HID, H)),
            full((1, HID)), full((1, OUT)), full((1, HID)), full((1, H)),
            full((1, 16)), full((1, 16)), full((D, H)), full((8, 3 * H)),
        ],
        out_specs=pl.BlockSpec((be_, EW), lambda i: (i, 0)),
        out_shape=jax.ShapeDtypeStruct((N_EDGES, EW), jnp.float32),
    )(g1, g2, edge_feat, r_feat, rx8,
      pk['W1'], pk['g'].reshape(1, -1), pk['be'].reshape(1, -1), pk['W2'],
      pv['W1'], pv['g'].reshape(1, -1), pv['be'].reshape(1, -1), pv['W2'],
      pk['b1'].reshape(1, -1), pk['b2'].reshape(1, -1),
      pv['b1'].reshape(1, -1), pv['b2'].reshape(1, -1),
      pew['W'].reshape(1, -1), jnp.broadcast_to(pew['b'].reshape(1, 1), (1, 16)),
      jnp.asarray(hsel), jnp.asarray(rsel))


def _phase_d_body(vals_hbm, dst_hbm, zeros_hbm, out_hbm, idx, buf, acc_sh, sem):
    c = lax.axis_index("c")
    s = lax.axis_index("s")
    wid = s * NC + c
    base0 = wid * EPW

    pltpu.sync_copy(zeros_hbm.at[pl.ds(0, NPT)], acc_sh.at[pl.ds(s * NPT, NPT)])
    plsc.subcore_barrier()

    nblk = EPW // BLK

    def fetch(i, sl):
        base = base0 + i * BLK
        pltpu.sync_copy(dst_hbm.at[pl.ds(base, BLK)], idx.at[sl])
        pltpu.async_copy(vals_hbm.at[pl.ds(base, BLK)], buf.at[sl], sem.at[sl])

    fetch(0, 0)

    def step(i, _):
        sl = lax.rem(i, 2)
        lax.cond(i + 1 < nblk,
                 lambda: fetch(i + 1, 1 - sl),
                 lambda: None)
        base = base0 + i * BLK
        pltpu.make_async_copy(
            vals_hbm.at[pl.ds(base, BLK)], buf.at[sl], sem.at[sl]).wait()
        pltpu.sync_copy(buf.at[sl], acc_sh.at[idx.at[sl]], add=True)
        return 0

    lax.fori_loop(0, nblk, step, 0)

    plsc.subcore_barrier()
    pltpu.sync_copy(acc_sh.at[pl.ds(s * NPT, NPT)],
                    out_hbm.at[c, pl.ds(s * NPT, NPT)])


def _phase_d(vals, dst, zeros):
    mesh = plsc.VectorSubcoreMesh(core_axis_name="c", subcore_axis_name="s")
    k = pl.kernel(
        _phase_d_body,
        out_type=jax.ShapeDtypeStruct((NC, N_PAD, EW), jnp.float32),
        mesh=mesh,
        scratch_types=[
            pltpu.VMEM((2, BLK), jnp.int32),
            pltpu.VMEM((2, BLK, EW), jnp.float32),
            pltpu.MemorySpace.VMEM_SHARED((N_PAD, EW), jnp.float32),
            pltpu.SemaphoreType.DMA((2,)),
        ],
    )
    return k(vals, dst, zeros)


def _phase_e_body(acc_ref, out_ref):
    a = acc_ref[0] + acc_ref[1]
    inv = 1.0 / ((a[:, :H] + 1e-16) * float(H))
    o0 = jnp.sum(a[:, H:2 * H] * inv, axis=-1, keepdims=True)
    o1 = jnp.sum(a[:, 2 * H:3 * H] * inv, axis=-1, keepdims=True)
    o2 = jnp.sum(a[:, 3 * H:4 * H] * inv, axis=-1, keepdims=True)
    out_ref[...] = jnp.concatenate([o0, o1, o2, o0 * 0.0], axis=-1)


def _phase_e(acc):
    bn = 2000
    grid = (N_NODES // bn,)
    return pl.pallas_call(
        _phase_e_body,
        grid=grid,
        in_specs=[pl.BlockSpec((NC, bn, EW), lambda i: (0, i, 0))],
        out_specs=pl.BlockSpec((bn, 4), lambda i: (i, 0)),
        out_shape=jax.ShapeDtypeStruct((N_NODES, 4), jnp.float32),
    )(acc)


def kernel(h, rel_x, r_feat, edge_feat, edge_index, params):
    src = edge_index[0]
    dst = edge_index[1]
    table = _phase_a(h, params['xq'])
    g1, g2 = _phase_b(table, h, dst, src)
    rx8 = jnp.concatenate(
        [rel_x, jnp.zeros((N_EDGES, 5), jnp.float32)], axis=1)
    ev = _phase_c(g1, g2, edge_feat, r_feat, rx8,
                  params['xk'], params['xv'], params['ew'])
    zeros = jnp.zeros((NPT, EW), jnp.float32)
    acc = _phase_d(ev, dst, zeros)
    out4 = _phase_e(acc)
    return out4[:, :3]
